# Initial kernel scaffold; baseline (speedup 1.0000x reference)
import functools

import jax
import jax.numpy as jnp
from jax import lax
from jax.experimental import pallas as pl
from jax.experimental.pallas import tpu as pltpu
from jax.experimental.pallas import tpu_sc as plsc

B, S, L = 1024, 50, 8
N = B * S * L
EMB = 64
NUM_EMB = 100002
BR = 256
GRID = -(-NUM_EMB // BR)
R_PAD = GRID * BR
NW = 32
PER_W = N // NW
VECS = PER_W // 16


def _rowsum_tc(table):
    def body(t_ref, o_ref):
        o_ref[...] = jnp.sum(t_ref[...], axis=1, keepdims=True)

    out = pl.pallas_call(
        body,
        grid=(GRID,),
        in_specs=[pl.BlockSpec((BR, EMB), lambda i: (i, 0))],
        out_specs=pl.BlockSpec((BR, 1), lambda i: (i, 0)),
        out_shape=jax.ShapeDtypeStruct((R_PAD, 1), jnp.float32),
    )(table)
    return out.reshape(R_PAD)


def _gather_sc(sums, ids_flat):
    mesh = plsc.VectorSubcoreMesh(core_axis_name="c", subcore_axis_name="s")

    @functools.partial(
        pl.kernel,
        mesh=mesh,
        out_type=jax.ShapeDtypeStruct((N,), jnp.float32),
        scratch_types=[
            pltpu.VMEM((R_PAD,), jnp.float32),
            pltpu.VMEM((PER_W,), jnp.int32),
            pltpu.VMEM((PER_W,), jnp.float32),
        ],
    )
    def k(sums_hbm, ids_hbm, out_hbm, sums_v, idx_v, out_v):
        wid = lax.axis_index("s") * 2 + lax.axis_index("c")
        base = wid * PER_W
        pltpu.sync_copy(sums_hbm, sums_v)
        pltpu.sync_copy(ids_hbm.at[pl.ds(base, PER_W)], idx_v)

        def body(i, carry):
            idx = idx_v[pl.ds(i * 16, 16)]
            out_v[pl.ds(i * 16, 16)] = plsc.load_gather(sums_v, [idx])
            return carry

        lax.fori_loop(0, VECS, body, 0)
        pltpu.sync_copy(out_v, out_hbm.at[pl.ds(base, PER_W)])

    return k(sums, ids_flat)


def kernel(zixing_ids, table):
    sums = _rowsum_tc(table)
    vals = _gather_sc(sums, zixing_ids.reshape(N))
    return vals.reshape(B, S, L, 1)


# speedup vs baseline: 4.0668x; 4.0668x over previous
import functools

import jax
import jax.numpy as jnp
from jax import lax
from jax.experimental import pallas as pl
from jax.experimental.pallas import tpu as pltpu
from jax.experimental.pallas import tpu_sc as plsc

B, S, L = 1024, 50, 8
N = B * S * L
EMB = 64
NUM_EMB = 100002
BR = 256
GRID = -(-NUM_EMB // BR)
R_PAD = GRID * BR
NW = 32
PER_W = N // NW
VECS = PER_W // 16


def _rowsum_tc(table):
    def body(t_ref, o_ref):
        o_ref[...] = jnp.sum(t_ref[...], axis=1, keepdims=True)

    out = pl.pallas_call(
        body,
        grid=(GRID,),
        in_specs=[pl.BlockSpec((BR, EMB), lambda i: (i, 0))],
        out_specs=pl.BlockSpec((BR, 1), lambda i: (i, 0)),
        out_shape=jax.ShapeDtypeStruct((R_PAD, 1), jnp.float32),
    )(table)
    return out.reshape(R_PAD)


def _gather_sc(sums, ids_flat):
    mesh = plsc.VectorSubcoreMesh(core_axis_name="c", subcore_axis_name="s")

    @functools.partial(
        pl.kernel,
        mesh=mesh,
        compiler_params=pltpu.CompilerParams(needs_layout_passes=False),
        out_type=jax.ShapeDtypeStruct((N,), jnp.float32),
        scratch_types=[
            pltpu.VMEM((R_PAD,), jnp.float32),
            pltpu.VMEM((PER_W,), jnp.int32),
            pltpu.VMEM((PER_W,), jnp.float32),
        ],
    )
    def k(sums_hbm, ids_hbm, out_hbm, sums_v, idx_v, out_v):
        wid = lax.axis_index("s") * 2 + lax.axis_index("c")
        base = wid * PER_W
        pltpu.sync_copy(sums_hbm, sums_v)
        pltpu.sync_copy(ids_hbm.at[pl.ds(base, PER_W)], idx_v)

        def body(i, carry):
            idx = idx_v[pl.ds(i * 16, 16)]
            out_v[pl.ds(i * 16, 16)] = plsc.load_gather(sums_v, [idx])
            return carry

        lax.fori_loop(0, VECS, body, 0)
        pltpu.sync_copy(out_v, out_hbm.at[pl.ds(base, PER_W)])

    return k(sums, ids_flat)


def kernel(zixing_ids, table):
    sums = _rowsum_tc(table)
    vals = _gather_sc(sums, zixing_ids.reshape(N))
    return vals.reshape(B, S, L, 1)


# speedup vs baseline: 23.4136x; 5.7573x over previous
import functools

import jax
import jax.numpy as jnp
from jax import lax
from jax.experimental import pallas as pl
from jax.experimental.pallas import tpu as pltpu
from jax.experimental.pallas import tpu_sc as plsc

B, S, L = 1024, 50, 8
N = B * S * L
EMB = 64
NUM_EMB = 100002
BC = 2048
GRID = -(-NUM_EMB // BC)
R_PAD = GRID * BC
NW = 32
PER_W = N // NW
VECS = PER_W // 16


def _rowsum_tc(table_t):
    def body(t_ref, o_ref):
        o_ref[...] = jnp.sum(t_ref[...], axis=0)

    return pl.pallas_call(
        body,
        grid=(GRID,),
        in_specs=[pl.BlockSpec((EMB, BC), lambda i: (0, i))],
        out_specs=pl.BlockSpec((BC,), lambda i: (i,)),
        out_shape=jax.ShapeDtypeStruct((R_PAD,), jnp.float32),
    )(table_t)


def _gather_sc(sums, ids_flat):
    mesh = plsc.VectorSubcoreMesh(core_axis_name="c", subcore_axis_name="s")

    @functools.partial(
        pl.kernel,
        mesh=mesh,
        compiler_params=pltpu.CompilerParams(needs_layout_passes=False),
        out_type=jax.ShapeDtypeStruct((N,), jnp.float32),
        scratch_types=[
            pltpu.VMEM((R_PAD,), jnp.float32),
            pltpu.VMEM((PER_W,), jnp.int32),
            pltpu.VMEM((PER_W,), jnp.float32),
        ],
    )
    def k(sums_hbm, ids_hbm, out_hbm, sums_v, idx_v, out_v):
        wid = lax.axis_index("s") * 2 + lax.axis_index("c")
        base = wid * PER_W
        pltpu.sync_copy(sums_hbm, sums_v)
        pltpu.sync_copy(ids_hbm.at[pl.ds(base, PER_W)], idx_v)

        def body(i, carry):
            idx = idx_v[pl.ds(i * 16, 16)]
            out_v[pl.ds(i * 16, 16)] = plsc.load_gather(sums_v, [idx])
            return carry

        lax.fori_loop(0, VECS, body, 0)
        pltpu.sync_copy(out_v, out_hbm.at[pl.ds(base, PER_W)])

    return k(sums, ids_flat)


def kernel(zixing_ids, table):
    ids_flat = zixing_ids.transpose(1, 2, 0).reshape(N)
    sums = _rowsum_tc(table.T)
    vals = _gather_sc(sums, ids_flat)
    return vals.reshape(S, L, B, 1).transpose(2, 0, 1, 3)
